# baseline (speedup 1.0000x reference)
import functools

import jax
import jax.numpy as jnp
from jax import lax
from jax.experimental import pallas as pl
from jax.experimental.pallas import tpu as pltpu
from jax.experimental.pallas import tpu_sc as plsc

_K = 2
_LANES = 16


def _router_body(x_ref, w_ref, aff_ref):
    logits = jnp.dot(x_ref[...], w_ref[...], preferred_element_type=jnp.float32)
    m = jnp.max(logits, axis=-1, keepdims=True)
    ex = jnp.exp(logits - m)
    aff_ref[...] = ex / jnp.sum(ex, axis=-1, keepdims=True)


def _make_sc_route(T):
    P = T * _K
    mesh = plsc.VectorSubcoreMesh(core_axis_name="c", subcore_axis_name="s")

    @functools.partial(
        pl.kernel,
        mesh=mesh,
        out_type=(
            jax.ShapeDtypeStruct((_LANES,), jnp.int32),
            jax.ShapeDtypeStruct((_LANES,), jnp.int32),
            jax.ShapeDtypeStruct((_LANES,), jnp.float32),
        ),
        scratch_types=[
            pltpu.VMEM((T * _LANES,), jnp.float32),
            pltpu.VMEM((T * _LANES,), jnp.float32),
            pltpu.VMEM((T * _LANES,), jnp.int32),
            pltpu.VMEM((_LANES,), jnp.float32),
            pltpu.VMEM((_LANES,), jnp.int32),
            pltpu.VMEM((_LANES,), jnp.int32),
            pltpu.VMEM((_LANES,), jnp.float32),
        ],
        compiler_params=pltpu.CompilerParams(needs_layout_passes=False),
    )
    def sc_route(aff_hbm, es_hbm, ts_hbm, vs_hbm,
                 aff_v, svals_v, sidx_v, pvals_v,
                 oe_v, ot_v, ov_v):
        c = lax.axis_index("c")
        s = lax.axis_index("s")

        @pl.when((c == 0) & (s == 0))
        def _():
            pltpu.sync_copy(aff_hbm, aff_v)
            lane = lax.iota(jnp.int32, _LANES)
            for t in range(T):
                row = aff_v[pl.ds(t * _LANES, _LANES)]
                sv, si = plsc.sort_key_val(row, lane, descending=True)
                svals_v[pl.ds(t * _LANES, _LANES)] = sv
                sidx_v[pl.ds(t * _LANES, _LANES)] = si
            pidx = jnp.where(
                lane < P,
                (lane // _K) * _LANES + (lane % _K),
                0,
            )
            keys = plsc.load_gather(sidx_v, [pidx])
            keys = jnp.where(lane < P, keys, 127)
            pv = plsc.load_gather(svals_v, [pidx])
            pvals_v[...] = jnp.where(lane < P, pv, 0.0)
            ks, ps = plsc.sort_key_val(keys, lane)
            oe_v[...] = ks
            ot_v[...] = ps // _K
            ov_v[...] = plsc.load_gather(pvals_v, [ps])
            pltpu.sync_copy(oe_v, es_hbm)
            pltpu.sync_copy(ot_v, ts_hbm)
            pltpu.sync_copy(ov_v, vs_hbm)

    return sc_route


def _mlp_body(e_ref, t_ref, v_ref, x_ref, g_ref, u_ref, d_ref, o_ref):
    j = pl.program_id(0)
    p = pl.program_id(1)
    T = o_ref.shape[0]

    @pl.when((j == 0) & (p == 0))
    def _():
        o_ref[...] = jnp.zeros_like(o_ref)

    xv = x_ref[0]
    g = jnp.dot(xv, g_ref[0], preferred_element_type=jnp.float32)
    u = jnp.dot(xv, u_ref[0], preferred_element_type=jnp.float32)
    a = g * jax.lax.logistic(g) * u
    part = jnp.dot(a, d_ref[0], preferred_element_type=jnp.float32)
    t = t_ref[p]
    scale = v_ref[p]
    rows = jax.lax.broadcasted_iota(jnp.int32, (T, 1), 0)
    o_ref[...] += jnp.where(rows == t, scale * part, 0.0)


def kernel(hidden_states, router_weight, gate_up_weights, down_weights):
    B, S, H = hidden_states.shape
    E = router_weight.shape[1]
    I = gate_up_weights.shape[2] // 2
    T = B * S
    P = T * _K
    x = hidden_states.reshape(T, H).astype(jnp.float32)

    aff = pl.pallas_call(
        _router_body,
        out_shape=jax.ShapeDtypeStruct((T, E), jnp.float32),
    )(x, router_weight.astype(jnp.float32))

    es, ts, vs = _make_sc_route(T)(aff.reshape(T * E))

    bI = 512
    J = I // bI

    grid_spec = pltpu.PrefetchScalarGridSpec(
        num_scalar_prefetch=3,
        grid=(J, P),
        in_specs=[
            pl.BlockSpec((1, 1, H), lambda j, p, e, t, v: (t[p], 0, 0)),
            pl.BlockSpec((1, H, bI), lambda j, p, e, t, v: (e[p], 0, j)),
            pl.BlockSpec((1, H, bI), lambda j, p, e, t, v: (e[p], 0, J + j)),
            pl.BlockSpec((1, bI, H), lambda j, p, e, t, v: (e[p], j, 0)),
        ],
        out_specs=pl.BlockSpec((T, H), lambda j, p, e, t, v: (0, 0)),
    )

    out = pl.pallas_call(
        _mlp_body,
        grid_spec=grid_spec,
        out_shape=jax.ShapeDtypeStruct((T, H), jnp.float32),
        compiler_params=pltpu.CompilerParams(
            dimension_semantics=("arbitrary", "arbitrary"),
        ),
    )(es, ts, vs, x.reshape(T, 1, H), gate_up_weights, gate_up_weights,
      down_weights)

    return out.reshape(B, S, H)
